# baseline (speedup 1.0000x reference)
import jax
import jax.numpy as jnp
from jax.experimental import pallas as pl

_CAP = 16


def _moe_kernel(x_ref, logits_ref, w1_ref, b1_ref, w2_ref, b2_ref, out_ref):
    e = pl.program_id(0)
    num_experts = pl.num_programs(0)

    logits = logits_ref[...]
    T = logits.shape[0]
    m = jnp.max(logits, axis=-1, keepdims=True)
    ex = jnp.exp(logits - m)
    probs = ex / jnp.sum(ex, axis=-1, keepdims=True)

    ids = jax.lax.broadcasted_iota(jnp.int32, probs.shape, 1)
    v1 = jnp.max(probs, axis=-1, keepdims=True)
    i1 = jnp.min(jnp.where(probs == v1, ids, num_experts), axis=-1,
                 keepdims=True)
    p2 = jnp.where(ids == i1, -jnp.inf, probs)
    v2 = jnp.max(p2, axis=-1, keepdims=True)
    i2 = jnp.min(jnp.where(p2 == v2, ids, num_experts), axis=-1,
                 keepdims=True)

    s = v1 + v2
    c = jnp.where(i1 == e, v1 / s, 0.0) + jnp.where(i2 == e, v2 / s, 0.0)

    sel = (c > 0.0).astype(jnp.float32)
    r_ids = jax.lax.broadcasted_iota(jnp.int32, (T, T), 0)
    c_ids = jax.lax.broadcasted_iota(jnp.int32, (T, T), 1)
    tri = (c_ids <= r_ids).astype(jnp.float32)
    pos = jnp.dot(tri, sel, preferred_element_type=jnp.float32,
                  precision=jax.lax.Precision.HIGHEST)
    count = pos[T - 1, 0]

    slot = jax.lax.broadcasted_iota(jnp.int32, (T, _CAP), 1).astype(jnp.float32)
    q = jnp.where((pos == slot + 1.0) & (sel > 0.0), 1.0, 0.0)

    x = x_ref[...]
    w1 = w1_ref[0]
    w2 = w2_ref[0]
    b1 = b1_ref[0]
    b2 = b2_ref[0]

    @pl.when(count <= _CAP)
    def _():
        xg = jnp.einsum('tp,td->pd', q, x,
                        preferred_element_type=jnp.float32,
                        precision=jax.lax.Precision.HIGHEST)
        sg = jnp.einsum('tp,tk->pk', q, c,
                        preferred_element_type=jnp.float32,
                        precision=jax.lax.Precision.HIGHEST)
        h = jnp.dot(xg, w1, preferred_element_type=jnp.float32) + b1
        a = jax.nn.gelu(h)
        y = jnp.dot(a, w2, preferred_element_type=jnp.float32) + b2
        z = sg * y
        out_ref[...] += jnp.dot(q, z, preferred_element_type=jnp.float32,
                                precision=jax.lax.Precision.HIGHEST)

    @pl.when(count > _CAP)
    def _():
        h = jnp.dot(x, w1, preferred_element_type=jnp.float32) + b1
        a = jax.nn.gelu(h)
        y = jnp.dot(a, w2, preferred_element_type=jnp.float32) + b2
        out_ref[...] += c * y


def _init_kernel(out_ref):
    out_ref[...] = jnp.zeros_like(out_ref)


def kernel(hidden_states, router_logits, w1, b1, w2, b2):
    T, D = hidden_states.shape
    E = router_logits.shape[1]
    F = w1.shape[2]
    b1 = b1.reshape(E, 1, F)
    b2 = b2.reshape(E, 1, D)

    def _wrapped(x_ref, logits_ref, w1_ref, b1_ref, w2_ref, b2_ref, out_ref):
        @pl.when(pl.program_id(0) == 0)
        def _():
            out_ref[...] = jnp.zeros_like(out_ref)

        _moe_kernel(x_ref, logits_ref, w1_ref, b1_ref, w2_ref, b2_ref, out_ref)

    return pl.pallas_call(
        _wrapped,
        grid=(E,),
        in_specs=[
            pl.BlockSpec((T, D), lambda e: (0, 0)),
            pl.BlockSpec((T, E), lambda e: (0, 0)),
            pl.BlockSpec((1, D, F), lambda e: (e, 0, 0)),
            pl.BlockSpec((1, 1, F), lambda e: (e, 0, 0)),
            pl.BlockSpec((1, F, D), lambda e: (e, 0, 0)),
            pl.BlockSpec((1, 1, D), lambda e: (e, 0, 0)),
        ],
        out_specs=pl.BlockSpec((T, D), lambda e: (0, 0)),
        out_shape=jax.ShapeDtypeStruct((T, D), jnp.float32),
    )(hidden_states, router_logits, w1, b1, w2, b2)


# speedup vs baseline: 1.0137x; 1.0137x over previous
import jax
import jax.numpy as jnp
from jax.experimental import pallas as pl


def _moe_kernel(x_ref, logits_ref, w1_ref, b1_ref, w2_ref, b2_ref, out_ref):
    e = pl.program_id(0)
    num_experts = pl.num_programs(0)

    logits = logits_ref[...]
    ids = jax.lax.broadcasted_iota(jnp.int32, logits.shape, 1)
    m1 = jnp.max(logits, axis=-1, keepdims=True)
    i1 = jnp.min(jnp.where(logits == m1, ids, num_experts), axis=-1,
                 keepdims=True)
    l2 = jnp.where(ids == i1, -jnp.inf, logits)
    m2 = jnp.max(l2, axis=-1, keepdims=True)
    i2 = jnp.min(jnp.where(l2 == m2, ids, num_experts), axis=-1,
                 keepdims=True)

    c1 = jax.nn.sigmoid(m1 - m2)
    c = jnp.where(i1 == e, c1, 0.0) + jnp.where(i2 == e, 1.0 - c1, 0.0)

    x = x_ref[...]
    h = jnp.dot(x, w1_ref[0], preferred_element_type=jnp.float32)
    h = h + b1_ref[0]
    a = jax.nn.gelu(h)
    y = jnp.dot(a, w2_ref[0], preferred_element_type=jnp.float32)
    y = y + b2_ref[0]
    contrib = c * y

    @pl.when(e == 0)
    def _():
        out_ref[...] = contrib

    @pl.when(e != 0)
    def _():
        out_ref[...] += contrib


def kernel(hidden_states, router_logits, w1, b1, w2, b2):
    T, D = hidden_states.shape
    E = router_logits.shape[1]
    F = w1.shape[2]
    b1 = b1.reshape(E, 1, F)
    b2 = b2.reshape(E, 1, D)

    return pl.pallas_call(
        _moe_kernel,
        grid=(E,),
        in_specs=[
            pl.BlockSpec((T, D), lambda e: (0, 0)),
            pl.BlockSpec((T, E), lambda e: (0, 0)),
            pl.BlockSpec((1, D, F), lambda e: (e, 0, 0)),
            pl.BlockSpec((1, 1, F), lambda e: (e, 0, 0)),
            pl.BlockSpec((1, F, D), lambda e: (e, 0, 0)),
            pl.BlockSpec((1, 1, D), lambda e: (e, 0, 0)),
        ],
        out_specs=pl.BlockSpec((T, D), lambda e: (0, 0)),
        out_shape=jax.ShapeDtypeStruct((T, D), jnp.float32),
    )(hidden_states, router_logits, w1, b1, w2, b2)


# speedup vs baseline: 1.1566x; 1.1409x over previous
import jax
import jax.numpy as jnp
from jax.experimental import pallas as pl
from jax.experimental.pallas import tpu as pltpu


def _top2(logits, num_experts):
    ids = jax.lax.broadcasted_iota(jnp.int32, logits.shape, 1)
    m1 = jnp.max(logits, axis=-1, keepdims=True)
    i1 = jnp.min(jnp.where(logits == m1, ids, num_experts), axis=-1,
                 keepdims=True)
    l2 = jnp.where(ids == i1, -jnp.inf, logits)
    m2 = jnp.max(l2, axis=-1, keepdims=True)
    i2 = jnp.min(jnp.where(l2 == m2, ids, num_experts), axis=-1,
                 keepdims=True)
    c1 = jax.nn.sigmoid(m1 - m2)
    return i1, i2, c1


def _routing_kernel(logits_ref, perm_ref):
    logits = logits_ref[...]
    T, E = logits.shape
    ids = jax.lax.broadcasted_iota(jnp.int32, logits.shape, 1)
    i1, i2, _ = _top2(logits, E)

    a = jnp.where((ids == i1) | (ids == i2), 1.0, 0.0)
    active_row = jnp.max(a, axis=0, keepdims=True)

    r_ids = jax.lax.broadcasted_iota(jnp.int32, (E, E), 0)
    c_ids = jax.lax.broadcasted_iota(jnp.int32, (E, E), 1)
    upper = (r_ids <= c_ids).astype(jnp.float32)
    pos_row = jnp.dot(active_row, upper,
                      preferred_element_type=jnp.float32)
    n_act = pos_row[0, E - 1]

    diag = r_ids == c_ids
    pos_col = jnp.sum(jnp.where(diag, jnp.broadcast_to(pos_row, (E, E)), 0.0),
                      axis=1, keepdims=True)
    act_col = jnp.sum(jnp.where(diag, jnp.broadcast_to(active_row, (E, E)),
                                0.0), axis=1, keepdims=True)

    slot_j = c_ids.astype(jnp.float32)
    g = jnp.where((act_col > 0.0) & (pos_col == slot_j + 1.0), 1.0, 0.0)
    e_row = jax.lax.broadcasted_iota(jnp.int32, (1, E), 1).astype(jnp.float32)
    perm_row = jnp.dot(e_row, g, preferred_element_type=jnp.float32)
    last_active = jnp.max(e_row * active_row, axis=1, keepdims=True)
    perm = jnp.where(e_row < n_act, perm_row, last_active)
    perm_ref[...] = perm.astype(jnp.int32)


def _moe_kernel(perm_ref, x_ref, logits_ref, w1_ref, b1_ref, w2_ref, b2_ref,
                out_ref):
    e = pl.program_id(0)
    num_experts = pl.num_programs(0)
    ep = perm_ref[e]

    logits = logits_ref[...]
    i1, i2, c1 = _top2(logits, num_experts)
    c = jnp.where(i1 == ep, c1, 0.0) + jnp.where(i2 == ep, 1.0 - c1, 0.0)

    def contrib():
        x = x_ref[...]
        h = jnp.dot(x, w1_ref[0], preferred_element_type=jnp.float32)
        h = h + b1_ref[0]
        a = jax.nn.gelu(h)
        y = jnp.dot(a, w2_ref[0], preferred_element_type=jnp.float32)
        y = y + b2_ref[0]
        return c * y

    @pl.when(e == 0)
    def _():
        out_ref[...] = contrib()

    @pl.when((e > 0) & (perm_ref[e] != perm_ref[e - 1]))
    def _():
        out_ref[...] += contrib()


def kernel(hidden_states, router_logits, w1, b1, w2, b2):
    T, D = hidden_states.shape
    E = router_logits.shape[1]
    F = w1.shape[2]
    b1 = b1.reshape(E, 1, F)
    b2 = b2.reshape(E, 1, D)

    perm = pl.pallas_call(
        _routing_kernel,
        out_shape=jax.ShapeDtypeStruct((1, E), jnp.int32),
    )(router_logits).reshape(E)

    grid_spec = pltpu.PrefetchScalarGridSpec(
        num_scalar_prefetch=1,
        grid=(E,),
        in_specs=[
            pl.BlockSpec((T, D), lambda e, p: (0, 0)),
            pl.BlockSpec((T, E), lambda e, p: (0, 0)),
            pl.BlockSpec((1, D, F), lambda e, p: (p[e], 0, 0)),
            pl.BlockSpec((1, 1, F), lambda e, p: (p[e], 0, 0)),
            pl.BlockSpec((1, F, D), lambda e, p: (p[e], 0, 0)),
            pl.BlockSpec((1, 1, D), lambda e, p: (p[e], 0, 0)),
        ],
        out_specs=pl.BlockSpec((T, D), lambda e, p: (0, 0)),
    )

    return pl.pallas_call(
        _moe_kernel,
        grid_spec=grid_spec,
        out_shape=jax.ShapeDtypeStruct((T, D), jnp.float32),
    )(perm, hidden_states, router_logits, w1, b1, w2, b2)


# speedup vs baseline: 1.1662x; 1.0083x over previous
import jax
import jax.numpy as jnp
from jax.experimental import pallas as pl
from jax.experimental.pallas import tpu as pltpu


def _routing_kernel(logits_ref, perm_ref, comb_ref):
    logits = logits_ref[...]
    T, E = logits.shape
    ids = jax.lax.broadcasted_iota(jnp.int32, logits.shape, 1)
    m1 = jnp.max(logits, axis=-1, keepdims=True)
    i1 = jnp.min(jnp.where(logits == m1, ids, E), axis=-1, keepdims=True)
    l2 = jnp.where(ids == i1, -jnp.inf, logits)
    m2 = jnp.max(l2, axis=-1, keepdims=True)
    i2 = jnp.min(jnp.where(l2 == m2, ids, E), axis=-1, keepdims=True)
    c1 = jax.nn.sigmoid(m1 - m2)

    tdiag = (jax.lax.broadcasted_iota(jnp.int32, (T, T), 0)
             == jax.lax.broadcasted_iota(jnp.int32, (T, T), 1))

    def to_row(v):
        return jnp.sum(jnp.where(tdiag, jnp.broadcast_to(v, (T, T)), 0.0),
                       axis=0, keepdims=True)

    i1_row = to_row(i1.astype(jnp.float32))
    i2_row = to_row(i2.astype(jnp.float32))
    c1_row = to_row(c1)
    e_col = jax.lax.broadcasted_iota(jnp.int32, (E, T), 0).astype(jnp.float32)
    comb = (jnp.where(e_col == i1_row, c1_row, 0.0)
            + jnp.where(e_col == i2_row, 1.0 - c1_row, 0.0))
    comb_ref[...] = comb

    a = jnp.where((ids == i1) | (ids == i2), 1.0, 0.0)
    active_row = jnp.max(a, axis=0, keepdims=True)

    r_ids = jax.lax.broadcasted_iota(jnp.int32, (E, E), 0)
    c_ids = jax.lax.broadcasted_iota(jnp.int32, (E, E), 1)
    upper = (r_ids <= c_ids).astype(jnp.float32)
    pos_row = jnp.dot(active_row, upper,
                      preferred_element_type=jnp.float32)
    n_act = pos_row[0, E - 1]

    diag = r_ids == c_ids
    pos_col = jnp.sum(jnp.where(diag, jnp.broadcast_to(pos_row, (E, E)), 0.0),
                      axis=1, keepdims=True)
    act_col = jnp.sum(jnp.where(diag, jnp.broadcast_to(active_row, (E, E)),
                                0.0), axis=1, keepdims=True)

    slot_j = c_ids.astype(jnp.float32)
    g = jnp.where((act_col > 0.0) & (pos_col == slot_j + 1.0), 1.0, 0.0)
    e_row = jax.lax.broadcasted_iota(jnp.int32, (1, E), 1).astype(jnp.float32)
    perm_row = jnp.dot(e_row, g, preferred_element_type=jnp.float32)
    last_active = jnp.max(e_row * active_row, axis=1, keepdims=True)
    perm = jnp.where(e_row < n_act, perm_row, last_active)
    perm_ref[...] = perm.astype(jnp.int32)


def _moe_kernel(perm_ref, x_ref, comb_ref, w1_ref, b1_ref, w2_ref, b2_ref,
                out_ref):
    e = pl.program_id(0)

    def contrib():
        c = comb_ref[0]
        x = x_ref[...]
        h = jnp.dot(x, w1_ref[0], preferred_element_type=jnp.float32)
        h = h + b1_ref[0]
        a = jax.nn.gelu(h)
        y = jnp.dot(a, w2_ref[0], preferred_element_type=jnp.float32)
        y = y + b2_ref[0]
        return c * y

    @pl.when(e == 0)
    def _():
        out_ref[...] = contrib()

    @pl.when((e > 0) & (perm_ref[e] != perm_ref[e - 1]))
    def _():
        out_ref[...] += contrib()


def kernel(hidden_states, router_logits, w1, b1, w2, b2):
    T, D = hidden_states.shape
    E = router_logits.shape[1]
    F = w1.shape[2]
    b1 = b1.reshape(E, 1, F)
    b2 = b2.reshape(E, 1, D)

    perm2d, comb = pl.pallas_call(
        _routing_kernel,
        out_shape=(
            jax.ShapeDtypeStruct((1, E), jnp.int32),
            jax.ShapeDtypeStruct((E, T), jnp.float32),
        ),
    )(router_logits)
    perm = perm2d.reshape(E)
    comb = comb.reshape(E, T, 1)

    grid_spec = pltpu.PrefetchScalarGridSpec(
        num_scalar_prefetch=1,
        grid=(E,),
        in_specs=[
            pl.BlockSpec((T, D), lambda e, p: (0, 0)),
            pl.BlockSpec((1, T, 1), lambda e, p: (p[e], 0, 0)),
            pl.BlockSpec((1, D, F), lambda e, p: (p[e], 0, 0)),
            pl.BlockSpec((1, 1, F), lambda e, p: (p[e], 0, 0)),
            pl.BlockSpec((1, F, D), lambda e, p: (p[e], 0, 0)),
            pl.BlockSpec((1, 1, D), lambda e, p: (p[e], 0, 0)),
        ],
        out_specs=pl.BlockSpec((T, D), lambda e, p: (0, 0)),
    )

    return pl.pallas_call(
        _moe_kernel,
        grid_spec=grid_spec,
        out_shape=jax.ShapeDtypeStruct((T, D), jnp.float32),
    )(perm, hidden_states, comb, w1, b1, w2, b2)
